# baseline (speedup 1.0000x reference)
import functools

import jax
import jax.numpy as jnp
import numpy as np
from jax import lax
from jax.experimental import pallas as pl
from jax.experimental.pallas import tpu as pltpu
from jax.experimental.pallas import tpu_sc as plsc

_VOCAB = 100000
_D = 64
_SEQ = 200
_BATCH = 4096

_NC = 2
_NS = 16
_NW = _NC * _NS
_SPW = _BATCH // _NW
_G = 40
_NG = _SEQ // _G


def _positional_encoding() -> jnp.ndarray:
    pos = np.arange(_SEQ, dtype=np.float64)[:, None]
    div = np.exp(np.arange(0, _D, 2, dtype=np.float64) * (-np.log(10000.0) / _D))
    pe = np.zeros((_SEQ, _D), dtype=np.float32)
    pe[:, 0::2] = np.sin(pos * div).astype(np.float32)
    pe[:, 1::2] = np.cos(pos * div).astype(np.float32)
    return jnp.asarray(pe)


_MESH = plsc.VectorSubcoreMesh(core_axis_name="c", subcore_axis_name="s")


@functools.partial(
    pl.kernel,
    mesh=_MESH,
    out_type=jax.ShapeDtypeStruct((_BATCH, _SEQ, _D), jnp.float32),
    scratch_types=[
        pltpu.VMEM((_SEQ,), jnp.int32),
        pltpu.VMEM((_SEQ,), jnp.int32),
        pltpu.VMEM((_SEQ, _D), jnp.float32),
        pltpu.VMEM((_SEQ, _D), jnp.float32),
        pltpu.VMEM((_SEQ, _D), jnp.float32),
        pltpu.SemaphoreType.DMA,
        pltpu.SemaphoreType.DMA,
        pltpu.SemaphoreType.DMA,
        pltpu.SemaphoreType.DMA,
    ],
    compiler_params=pltpu.CompilerParams(use_tc_tiling_on_sc=False),
)
def _embed(x_hbm, table_hbm, pe_hbm, out_hbm,
           idx0, idx1, g0, g1, pe_v, sg0, sg1, so0, so1):
    wid = lax.axis_index("s") * _NC + lax.axis_index("c")
    base = wid * _SPW
    pltpu.sync_copy(pe_hbm, pe_v)
    slots = ((idx0, g0, sg0, so0), (idx1, g1, sg1, so1))

    def fire_gathers(idx_b, g_b, sem):
        for g in range(_NG):
            pltpu.async_copy(
                table_hbm.at[idx_b.at[pl.ds(g * _G, _G)]],
                g_b.at[pl.ds(g * _G, _G)],
                sem,
            )

    def wait_gathers(idx_b, g_b, sem):
        for g in range(_NG):
            pltpu.make_async_copy(
                table_hbm.at[idx_b.at[pl.ds(g * _G, _G)]],
                g_b.at[pl.ds(g * _G, _G)],
                sem,
            ).wait()

    def wait_out(cc, g_b, sem):
        pltpu.make_async_copy(g_b, out_hbm.at[base + cc], sem).wait()

    pltpu.sync_copy(x_hbm.at[base], idx0)
    pltpu.sync_copy(x_hbm.at[base + 1], idx1)
    fire_gathers(idx0, g0, sg0)

    def body(ci, carry):
        for b in range(2):
            idx_b, g_b, sg_b, so_b = slots[b]
            idx_n, g_n, sg_n, so_n = slots[1 - b]
            cc = ci * 2 + b
            wait_gathers(idx_b, g_b, sg_b)

            def add_pe(q, _):
                for h in range(2):
                    s = 2 * q + h
                    for j in range(4):
                        d = pl.ds(j * 16, 16)
                        g_b[s, d] = g_b[s, d] + pe_v[s, d]
                return _

            lax.fori_loop(0, _SEQ // 2, add_pe, 0)
            pltpu.async_copy(g_b, out_hbm.at[base + cc], so_b)

            @pl.when(cc + 2 < _SPW)
            def _():
                pltpu.sync_copy(x_hbm.at[base + cc + 2], idx_b)

            @pl.when(cc + 1 < _SPW)
            def _():
                @pl.when(cc >= 1)
                def _():
                    wait_out(cc - 1, g_n, so_n)

                fire_gathers(idx_n, g_n, sg_n)

        return carry

    lax.fori_loop(0, _SPW // 2, body, 0)
    wait_out(_SPW - 2, g0, so0)
    wait_out(_SPW - 1, g1, so1)


def kernel(x, table):
    return _embed(x.astype(jnp.int32), table, _positional_encoding())


# speedup vs baseline: 1.1293x; 1.1293x over previous
import functools

import jax
import jax.numpy as jnp
import numpy as np
from jax import lax
from jax.experimental import pallas as pl
from jax.experimental.pallas import tpu as pltpu
from jax.experimental.pallas import tpu_sc as plsc

_VOCAB = 100000
_D = 64
_DP = 128
_SEQ = 200
_BATCH = 4096

_NC = 2
_NS = 16
_NW = _NC * _NS
_SPW = _BATCH // _NW
_G = 40
_NG = _SEQ // _G
_H = _SEQ // 2


def _positional_encoding_packed() -> jnp.ndarray:
    pos = np.arange(_SEQ, dtype=np.float64)[:, None]
    div = np.exp(np.arange(0, _D, 2, dtype=np.float64) * (-np.log(10000.0) / _D))
    pe = np.zeros((_SEQ, _D), dtype=np.float32)
    pe[:, 0::2] = np.sin(pos * div).astype(np.float32)
    pe[:, 1::2] = np.cos(pos * div).astype(np.float32)
    return jnp.asarray(np.concatenate([pe[:_H], pe[_H:]], axis=1))


_MESH = plsc.VectorSubcoreMesh(core_axis_name="c", subcore_axis_name="s")


@functools.partial(
    pl.kernel,
    mesh=_MESH,
    out_type=jax.ShapeDtypeStruct((_BATCH, _SEQ, _D), jnp.float32),
    scratch_types=[
        pltpu.VMEM((_SEQ,), jnp.int32),
        pltpu.VMEM((_SEQ,), jnp.int32),
        pltpu.VMEM((_SEQ, _DP), jnp.float32),
        pltpu.VMEM((_SEQ, _DP), jnp.float32),
        pltpu.VMEM((_SEQ, _D), jnp.float32),
        pltpu.VMEM((_SEQ, _D), jnp.float32),
        pltpu.VMEM((_H, _DP), jnp.float32),
        pltpu.SemaphoreType.DMA,
        pltpu.SemaphoreType.DMA,
        pltpu.SemaphoreType.DMA,
        pltpu.SemaphoreType.DMA,
    ],
    compiler_params=pltpu.CompilerParams(use_tc_tiling_on_sc=True),
)
def _embed(x_hbm, table_hbm, pe_hbm, out_hbm,
           idx0, idx1, g0, g1, w0, w1, pe_v, sg0, sg1, so0, so1):
    wid = lax.axis_index("s") * _NC + lax.axis_index("c")
    base = wid * _SPW
    pltpu.sync_copy(pe_hbm, pe_v)
    slots = ((idx0, g0, w0, sg0, so0), (idx1, g1, w1, sg1, so1))

    def fire_gathers(idx_b, g_b, sem):
        for g in range(_NG):
            pltpu.async_copy(
                table_hbm.at[idx_b.at[pl.ds(g * _G, _G)]],
                g_b.at[pl.ds(g * _G, _G)],
                sem,
            )

    def wait_gathers(idx_b, g_b, sem):
        for g in range(_NG):
            pltpu.make_async_copy(
                table_hbm.at[idx_b.at[pl.ds(g * _G, _G)]],
                g_b.at[pl.ds(g * _G, _G)],
                sem,
            ).wait()

    def wait_out(cc, w_b, sem):
        pltpu.make_async_copy(w_b, out_hbm.at[base + cc], sem).wait()

    pltpu.sync_copy(x_hbm.at[base], idx0)
    pltpu.sync_copy(x_hbm.at[base + 1], idx1)
    fire_gathers(idx0, g0, sg0)

    def body(ci, carry):
        for b in range(2):
            idx_b, g_b, w_b, sg_b, so_b = slots[b]
            idx_n, g_n, w_n, sg_n, so_n = slots[1 - b]
            cc = ci * 2 + b
            wait_gathers(idx_b, g_b, sg_b)

            def add_pe(q, _):
                for h in range(2):
                    s = 2 * q + h
                    for j in range(4):
                        d = pl.ds(j * 16, 16)
                        w_b[s, d] = g_b[s, d] + pe_v[s, d]
                        dh = pl.ds(_D + j * 16, 16)
                        w_b[s + _H, d] = g_b[s + _H, d] + pe_v[s, dh]
                return _

            lax.fori_loop(0, _H // 2, add_pe, 0)
            pltpu.async_copy(w_b, out_hbm.at[base + cc], so_b)

            @pl.when(cc + 2 < _SPW)
            def _():
                pltpu.sync_copy(x_hbm.at[base + cc + 2], idx_b)

            @pl.when(cc + 1 < _SPW)
            def _():
                @pl.when(cc >= 1)
                def _():
                    wait_out(cc - 1, w_n, so_n)

                fire_gathers(idx_n, g_n, sg_n)

        return carry

    lax.fori_loop(0, _SPW // 2, body, 0)
    wait_out(_SPW - 2, w0, so0)
    wait_out(_SPW - 1, w1, so1)


def kernel(x, table):
    tablep = jnp.pad(table, ((0, 0), (0, _DP - _D)))
    return _embed(x.astype(jnp.int32), tablep, _positional_encoding_packed())
